# Initial kernel scaffold; baseline (speedup 1.0000x reference)
import jax
import jax.numpy as jnp
from jax.experimental import pallas as pl


def kernel(user_id, movie_id, user_rating, user_table, movie_table, W0, b0, W1, b1, W2, b2):
    raise NotImplementedError("write your pallas kernel here")


# speedup vs baseline: 1.3558x; 1.3558x over previous
import functools

import jax
import jax.numpy as jnp
from jax import lax
from jax.experimental import pallas as pl
from jax.experimental.pallas import tpu as pltpu
from jax.experimental.pallas import tpu_sc as plsc

B = 16384
EMB = 32

_info = plsc.get_sparse_core_info()
_NC, _NS = _info.num_cores, _info.num_subcores
_NW = _NC * _NS
_BPW = B // _NW

_mesh = plsc.VectorSubcoreMesh(core_axis_name="c", subcore_axis_name="s")


@functools.partial(
    pl.kernel,
    mesh=_mesh,
    compiler_params=pltpu.CompilerParams(use_tc_tiling_on_sc=False),
    out_type=(
        jax.ShapeDtypeStruct((B, EMB), jnp.float32),
        jax.ShapeDtypeStruct((B, EMB), jnp.float32),
    ),
    scratch_types=[
        pltpu.VMEM((_BPW,), jnp.int32),
        pltpu.VMEM((_BPW, EMB), jnp.float32),
        pltpu.VMEM((_BPW,), jnp.int32),
        pltpu.VMEM((_BPW, EMB), jnp.float32),
        pltpu.SemaphoreType.DMA,
        pltpu.SemaphoreType.DMA,
    ],
)
def _sc_gather(uid_hbm, mid_hbm, utab_hbm, mtab_hbm, uout_hbm, mout_hbm,
               uidx_v, urows_v, midx_v, mrows_v, usem, msem):
    wid = lax.axis_index("s") * _NC + lax.axis_index("c")
    base = wid * _BPW
    pltpu.sync_copy(uid_hbm.at[pl.ds(base, _BPW)], uidx_v)
    pltpu.sync_copy(mid_hbm.at[pl.ds(base, _BPW)], midx_v)
    ucp = pltpu.async_copy(utab_hbm.at[uidx_v], urows_v, usem)
    mcp = pltpu.async_copy(mtab_hbm.at[midx_v], mrows_v, msem)
    ucp.wait()
    mcp.wait()
    pltpu.sync_copy(urows_v, uout_hbm.at[pl.ds(base, _BPW)])
    pltpu.sync_copy(mrows_v, mout_hbm.at[pl.ds(base, _BPW)])


_NB = 16
_BLK = B // _NB


def _mlp_body(u_ref, m_ref, r_ref, w0u_ref, w0m_ref, b0_ref, w1_ref, b1_ref,
              w2t_ref, b2_ref, out_ref):
    u = u_ref[...]
    m = m_ref[...]
    h = jnp.dot(u, w0u_ref[...], preferred_element_type=jnp.float32)
    h = h + jnp.dot(m, w0m_ref[...], preferred_element_type=jnp.float32)
    h = jnp.maximum(h + b0_ref[...], 0.0)
    h = jnp.dot(h, w1_ref[...], preferred_element_type=jnp.float32)
    h = jnp.maximum(h + b1_ref[...], 0.0)
    o = jnp.sum(h * w2t_ref[...], axis=1) + b2_ref[0]
    d = r_ref[0, 0, :] - o
    part = jnp.sum(d * d)

    @pl.when(pl.program_id(0) == 0)
    def _():
        out_ref[0] = 0.0

    out_ref[0] = out_ref[0] + part


def _mlp_loss(u, m, r3, w0u, w0m, b0r, w1, b1r, w2t, b2):
    return pl.pallas_call(
        _mlp_body,
        grid=(_NB,),
        in_specs=[
            pl.BlockSpec((_BLK, EMB), lambda i: (i, 0)),
            pl.BlockSpec((_BLK, EMB), lambda i: (i, 0)),
            pl.BlockSpec((1, 1, _BLK), lambda i: (i, 0, 0)),
            pl.BlockSpec((EMB, 256), lambda i: (0, 0)),
            pl.BlockSpec((EMB, 256), lambda i: (0, 0)),
            pl.BlockSpec((1, 256), lambda i: (0, 0)),
            pl.BlockSpec((256, 64), lambda i: (0, 0)),
            pl.BlockSpec((1, 64), lambda i: (0, 0)),
            pl.BlockSpec((1, 64), lambda i: (0, 0)),
            pl.BlockSpec(memory_space=pltpu.SMEM),
        ],
        out_specs=pl.BlockSpec(memory_space=pltpu.SMEM),
        out_shape=jax.ShapeDtypeStruct((1,), jnp.float32),
    )(u, m, r3, w0u, w0m, b0r, w1, b1r, w2t, b2)


def kernel(user_id, movie_id, user_rating, user_table, movie_table,
           W0, b0, W1, b1, W2, b2):
    uid = user_id.astype(jnp.int32)
    mid = movie_id.astype(jnp.int32)
    u, m = _sc_gather(uid, mid, user_table, movie_table)
    loss = _mlp_loss(
        u, m,
        user_rating.reshape(_NB, 1, _BLK),
        W0[:EMB], W0[EMB:],
        b0.reshape(1, 256),
        W1,
        b1.reshape(1, 64),
        W2.reshape(1, 64),
        b2,
    )
    return loss[0] / jnp.float32(B)


# speedup vs baseline: 1.3563x; 1.0003x over previous
import functools

import jax
import jax.numpy as jnp
from jax import lax
from jax.experimental import pallas as pl
from jax.experimental.pallas import tpu as pltpu
from jax.experimental.pallas import tpu_sc as plsc

B = 16384
EMB = 32

_info = plsc.get_sparse_core_info()
_NC, _NS = _info.num_cores, _info.num_subcores
_NW = _NC * _NS
_BPW = B // _NW
_CH = 128
_NCH = _BPW // _CH

_mesh = plsc.VectorSubcoreMesh(core_axis_name="c", subcore_axis_name="s")


@functools.partial(
    pl.kernel,
    mesh=_mesh,
    out_type=(
        jax.ShapeDtypeStruct((B, 128), jnp.float32),
        jax.ShapeDtypeStruct((B, 128), jnp.float32),
    ),
    scratch_types=[
        pltpu.VMEM((_NCH, _CH), jnp.int32),
        pltpu.VMEM((_NCH, _CH), jnp.int32),
        pltpu.VMEM((_CH, 128), jnp.float32),
        pltpu.VMEM((_CH, 128), jnp.float32),
        pltpu.SemaphoreType.DMA,
        pltpu.SemaphoreType.DMA,
    ],
)
def _sc_gather(uid_hbm, mid_hbm, utab_hbm, mtab_hbm, uout_hbm, mout_hbm,
               uidx_v, midx_v, urows_v, mrows_v, usem, msem):
    wid = lax.axis_index("s") * _NC + lax.axis_index("c")
    base = wid * _BPW
    pltpu.sync_copy(uid_hbm.at[pl.ds(wid * _NCH, _NCH)], uidx_v)
    pltpu.sync_copy(mid_hbm.at[pl.ds(wid * _NCH, _NCH)], midx_v)
    for j in range(_NCH):
        for k in range(_CH // 16):
            s = pl.ds(k * 16, 16)
            uidx_v[j, s] = lax.shift_right_logical(uidx_v[j, s], 2)
            midx_v[j, s] = lax.shift_right_logical(midx_v[j, s], 2)
    for j in range(_NCH):
        ucp = pltpu.async_copy(utab_hbm.at[uidx_v.at[j]], urows_v, usem)
        mcp = pltpu.async_copy(mtab_hbm.at[midx_v.at[j]], mrows_v, msem)
        ucp.wait()
        pltpu.sync_copy(urows_v, uout_hbm.at[pl.ds(base + j * _CH, _CH)])
        mcp.wait()
        pltpu.sync_copy(mrows_v, mout_hbm.at[pl.ds(base + j * _CH, _CH)])


_NB = 16
_BLK = B // _NB


def _mlp_body(ut_ref, mt_ref, uph_ref, mph_ref, r_ref, w0u_ref, w0m_ref,
              b0_ref, w1_ref, b1_ref, w2t_ref, b2_ref, out_ref):
    lane = lax.broadcasted_iota(jnp.int32, (_BLK, 128), 1) >> 5
    umask = (lane == (uph_ref[0, 0, :] & 3)[:, None]).astype(jnp.float32)
    mmask = (lane == (mph_ref[0, 0, :] & 3)[:, None]).astype(jnp.float32)
    h = jnp.dot(ut_ref[...] * umask, w0u_ref[...],
                preferred_element_type=jnp.float32)
    h = h + jnp.dot(mt_ref[...] * mmask, w0m_ref[...],
                    preferred_element_type=jnp.float32)
    h = jnp.maximum(h + b0_ref[...], 0.0)
    h = jnp.dot(h, w1_ref[...], preferred_element_type=jnp.float32)
    h = jnp.maximum(h + b1_ref[...], 0.0)
    o = jnp.sum(h * w2t_ref[...], axis=1) + b2_ref[0]
    d = r_ref[0, 0, :] - o
    part = jnp.sum(d * d)

    @pl.when(pl.program_id(0) == 0)
    def _():
        out_ref[0] = 0.0

    out_ref[0] = out_ref[0] + part


def _mlp_loss(ut, mt, uph3, mph3, r3, w0u, w0m, b0r, w1, b1r, w2t, b2):
    return pl.pallas_call(
        _mlp_body,
        grid=(_NB,),
        in_specs=[
            pl.BlockSpec((_BLK, 128), lambda i: (i, 0)),
            pl.BlockSpec((_BLK, 128), lambda i: (i, 0)),
            pl.BlockSpec((1, 1, _BLK), lambda i: (i, 0, 0)),
            pl.BlockSpec((1, 1, _BLK), lambda i: (i, 0, 0)),
            pl.BlockSpec((1, 1, _BLK), lambda i: (i, 0, 0)),
            pl.BlockSpec((128, 256), lambda i: (0, 0)),
            pl.BlockSpec((128, 256), lambda i: (0, 0)),
            pl.BlockSpec((1, 256), lambda i: (0, 0)),
            pl.BlockSpec((256, 64), lambda i: (0, 0)),
            pl.BlockSpec((1, 64), lambda i: (0, 0)),
            pl.BlockSpec((1, 64), lambda i: (0, 0)),
            pl.BlockSpec(memory_space=pltpu.SMEM),
        ],
        out_specs=pl.BlockSpec(memory_space=pltpu.SMEM),
        out_shape=jax.ShapeDtypeStruct((1,), jnp.float32),
    )(ut, mt, uph3, mph3, r3, w0u, w0m, b0r, w1, b1r, w2t, b2)


def kernel(user_id, movie_id, user_rating, user_table, movie_table,
           W0, b0, W1, b1, W2, b2):
    uid = user_id.astype(jnp.int32)
    mid = movie_id.astype(jnp.int32)
    ut, mt = _sc_gather(
        uid.reshape(B // _CH, _CH),
        mid.reshape(B // _CH, _CH),
        user_table.reshape(-1, 128),
        movie_table.reshape(-1, 128),
    )
    loss = _mlp_loss(
        ut, mt,
        uid.reshape(_NB, 1, _BLK),
        mid.reshape(_NB, 1, _BLK),
        user_rating.reshape(_NB, 1, _BLK),
        jnp.tile(W0[:EMB], (4, 1)),
        jnp.tile(W0[EMB:], (4, 1)),
        b0.reshape(1, 256),
        W1,
        b1.reshape(1, 64),
        W2.reshape(1, 64),
        b2,
    )
    return loss[0] / jnp.float32(B)


# speedup vs baseline: 2.5551x; 1.8839x over previous
import functools

import jax
import jax.numpy as jnp
from jax import lax
from jax.experimental import pallas as pl
from jax.experimental.pallas import tpu as pltpu
from jax.experimental.pallas import tpu_sc as plsc

B = 16384
EMB = 32

_info = plsc.get_sparse_core_info()
_NC, _NS = _info.num_cores, _info.num_subcores
_NW = _NC * _NS
_BPW = B // _NW
_G = 16
_NG = _BPW // _G

_mesh = plsc.VectorSubcoreMesh(core_axis_name="c", subcore_axis_name="s")


@functools.partial(
    pl.kernel,
    mesh=_mesh,
    out_type=(
        jax.ShapeDtypeStruct((B, EMB), jnp.float32),
        jax.ShapeDtypeStruct((B, EMB), jnp.float32),
    ),
    scratch_types=[
        pltpu.VMEM((_BPW,), jnp.int32),
        pltpu.VMEM((_BPW,), jnp.int32),
        pltpu.VMEM((_G, 8, EMB), jnp.float32),
        pltpu.VMEM((_G, 8, EMB), jnp.float32),
        pltpu.VMEM((_G, EMB), jnp.float32),
        pltpu.VMEM((_G, EMB), jnp.float32),
        pltpu.SemaphoreType.DMA,
        pltpu.SemaphoreType.DMA,
    ],
)
def _sc_gather(uid_hbm, mid_hbm, utab_hbm, mtab_hbm, uout_hbm, mout_hbm,
               uidx_v, midx_v, ubuf_v, mbuf_v, urows_v, mrows_v, usem, msem):
    wid = lax.axis_index("s") * _NC + lax.axis_index("c")
    base = wid * _BPW
    pltpu.sync_copy(uid_hbm.at[pl.ds(base, _BPW)], uidx_v)
    pltpu.sync_copy(mid_hbm.at[pl.ds(base, _BPW)], midx_v)

    def group(g, carry):
        uids = uidx_v[pl.ds(g * _G, _G)]
        mids = midx_v[pl.ds(g * _G, _G)]
        utids = lax.shift_right_logical(uids, 3)
        mtids = lax.shift_right_logical(mids, 3)
        uphs = lax.bitwise_and(uids, 7)
        mphs = lax.bitwise_and(mids, 7)
        ucps = [
            pltpu.async_copy(utab_hbm.at[pl.ds(utids[k], 1)],
                             ubuf_v.at[pl.ds(k, 1)], usem)
            for k in range(_G)
        ]
        mcps = [
            pltpu.async_copy(mtab_hbm.at[pl.ds(mtids[k], 1)],
                             mbuf_v.at[pl.ds(k, 1)], msem)
            for k in range(_G)
        ]
        for cp in ucps:
            cp.wait()
        for k in range(_G):
            ph = uphs[k]
            urows_v[k, pl.ds(0, 16)] = ubuf_v[k, ph, pl.ds(0, 16)]
            urows_v[k, pl.ds(16, 16)] = ubuf_v[k, ph, pl.ds(16, 16)]
        pltpu.sync_copy(urows_v, uout_hbm.at[pl.ds(base + g * _G, _G)])
        for cp in mcps:
            cp.wait()
        for k in range(_G):
            ph = mphs[k]
            mrows_v[k, pl.ds(0, 16)] = mbuf_v[k, ph, pl.ds(0, 16)]
            mrows_v[k, pl.ds(16, 16)] = mbuf_v[k, ph, pl.ds(16, 16)]
        pltpu.sync_copy(mrows_v, mout_hbm.at[pl.ds(base + g * _G, _G)])
        return carry

    lax.fori_loop(0, _NG, group, 0)


_NB = 16
_BLK = B // _NB


def _mlp_body(u_ref, m_ref, r_ref, w0u_ref, w0m_ref, b0_ref, w1_ref, b1_ref,
              w2t_ref, b2_ref, out_ref):
    u = u_ref[...]
    m = m_ref[...]
    h = jnp.dot(u, w0u_ref[...], preferred_element_type=jnp.float32)
    h = h + jnp.dot(m, w0m_ref[...], preferred_element_type=jnp.float32)
    h = jnp.maximum(h + b0_ref[...], 0.0)
    h = jnp.dot(h, w1_ref[...], preferred_element_type=jnp.float32)
    h = jnp.maximum(h + b1_ref[...], 0.0)
    o = jnp.sum(h * w2t_ref[...], axis=1) + b2_ref[0]
    d = r_ref[0, 0, :] - o
    part = jnp.sum(d * d)

    @pl.when(pl.program_id(0) == 0)
    def _():
        out_ref[0] = 0.0

    out_ref[0] = out_ref[0] + part


def _mlp_loss(u, m, r3, w0u, w0m, b0r, w1, b1r, w2t, b2):
    return pl.pallas_call(
        _mlp_body,
        grid=(_NB,),
        in_specs=[
            pl.BlockSpec((_BLK, EMB), lambda i: (i, 0)),
            pl.BlockSpec((_BLK, EMB), lambda i: (i, 0)),
            pl.BlockSpec((1, 1, _BLK), lambda i: (i, 0, 0)),
            pl.BlockSpec((EMB, 256), lambda i: (0, 0)),
            pl.BlockSpec((EMB, 256), lambda i: (0, 0)),
            pl.BlockSpec((1, 256), lambda i: (0, 0)),
            pl.BlockSpec((256, 64), lambda i: (0, 0)),
            pl.BlockSpec((1, 64), lambda i: (0, 0)),
            pl.BlockSpec((1, 64), lambda i: (0, 0)),
            pl.BlockSpec(memory_space=pltpu.SMEM),
        ],
        out_specs=pl.BlockSpec(memory_space=pltpu.SMEM),
        out_shape=jax.ShapeDtypeStruct((1,), jnp.float32),
    )(u, m, r3, w0u, w0m, b0r, w1, b1r, w2t, b2)


def kernel(user_id, movie_id, user_rating, user_table, movie_table,
           W0, b0, W1, b1, W2, b2):
    uid = user_id.astype(jnp.int32)
    mid = movie_id.astype(jnp.int32)
    u, m = _sc_gather(
        uid, mid,
        user_table.reshape(-1, 8, EMB),
        movie_table.reshape(-1, 8, EMB),
    )
    loss = _mlp_loss(
        u, m,
        user_rating.reshape(_NB, 1, _BLK),
        W0[:EMB], W0[EMB:],
        b0.reshape(1, 256),
        W1,
        b1.reshape(1, 64),
        W2.reshape(1, 64),
        b2,
    )
    return loss[0] / jnp.float32(B)
